# Initial kernel scaffold; baseline (speedup 1.0000x reference)
import jax
import jax.numpy as jnp
from jax.experimental import pallas as pl


def kernel(embeddings, labels):
    raise NotImplementedError("write your pallas kernel here")


# speedup vs baseline: 20.2981x; 20.2981x over previous
import functools

import jax
import jax.numpy as jnp
from jax.experimental import pallas as pl
from jax.experimental.pallas import tpu as pltpu

_NUM_STEPS = 100
_MARGIN = 0.1


def _hist_body(emb_ref, lab_row_ref, lab_col_ref, out_ref,
               nemb_ref, hacc_ref, *, block_rows, batch):
    i = pl.program_id(0)
    nsteps = pl.num_programs(0)

    @pl.when(i == 0)
    def _init():
        x = emb_ref[...]
        norm = jnp.sqrt(jnp.sum(x * x, axis=1, keepdims=True))
        nemb_ref[...] = x / jnp.maximum(norm, 1e-12)
        hacc_ref[...] = jnp.zeros_like(hacc_ref)

    rows = nemb_ref[pl.ds(i * block_rows, block_rows), :]
    sim = jax.lax.dot_general(
        rows, nemb_ref[...],
        dimension_numbers=(((1,), (1,)), ((), ())),
        preferred_element_type=jnp.float32,
    )

    lab_r = lab_col_ref[pl.ds(i * block_rows, block_rows), :]
    lab_c = lab_row_ref[...]
    eq = lab_r == lab_c
    col_ids = jax.lax.broadcasted_iota(jnp.int32, sim.shape, 1)
    row_ids = jax.lax.broadcasted_iota(jnp.int32, sim.shape, 0) + i * block_rows
    isdiag = col_ids == row_ids

    idx = jnp.clip(((sim + 1.0) * (_NUM_STEPS / 2.0)).astype(jnp.int32),
                   0, _NUM_STEPS - 1)
    idxm = jnp.where(eq, idx, 255)
    diag_v = jnp.sum(jnp.where(isdiag, sim, 0.0), axis=1, keepdims=True)
    diag_idx = jnp.sum(jnp.where(isdiag, idx, 0), axis=1, keepdims=True)

    lane = jax.lax.broadcasted_iota(jnp.int32, (1, 128), 1)
    ha = hacc_ref[0:1, :]
    he = hacc_ref[1:2, :]
    hd = hacc_ref[2:3, :]
    for b in range(_NUM_STEPS):
        ha = ha + jnp.where(lane == b, jnp.sum((idx == b).astype(jnp.float32)), 0.0)
        he = he + jnp.where(lane == b, jnp.sum((idxm == b).astype(jnp.float32)), 0.0)
        hd = hd + jnp.where(lane == b, jnp.sum((diag_idx == b).astype(jnp.float32)), 0.0)
    hacc_ref[0:1, :] = ha
    hacc_ref[1:2, :] = he
    hacc_ref[2:3, :] = hd

    sums = hacc_ref[3:4, :]
    sums = sums + jnp.where(lane == 0, jnp.sum(sim), 0.0)
    sums = sums + jnp.where(lane == 1, jnp.sum(jnp.where(eq, sim, 0.0)), 0.0)
    sums = sums + jnp.where(lane == 2, jnp.sum(eq.astype(jnp.float32)), 0.0)
    sums = sums + jnp.where(lane == 3, jnp.sum(diag_v), 0.0)
    hacc_ref[3:4, :] = sums

    @pl.when(i == nsteps - 1)
    def _finish():
        hist_all = hacc_ref[0:1, :]
        hist_eq = hacc_ref[1:2, :]
        hist_diag = hacc_ref[2:3, :]
        pos_hist = hist_eq - hist_diag
        neg_hist = hist_all - hist_eq
        pos_hist = pos_hist / (jnp.sum(pos_hist) + 1e-16)
        neg_hist = neg_hist / (jnp.sum(neg_hist) + 1e-16)
        overlap = jnp.sum(jnp.minimum(pos_hist, neg_hist))

        s = hacc_ref[3:4, :]
        tot_sum = jnp.sum(jnp.where(lane == 0, s, 0.0))
        eq_sum = jnp.sum(jnp.where(lane == 1, s, 0.0))
        eq_count = jnp.sum(jnp.where(lane == 2, s, 0.0))
        diag_sum = jnp.sum(jnp.where(lane == 3, s, 0.0))
        bf = jnp.float32(batch)
        pos_mean = (eq_sum - diag_sum) / (eq_count - bf)
        neg_mean = (tot_sum - eq_sum) / (bf * bf - eq_count)
        loss = overlap + jnp.maximum(_MARGIN - (pos_mean - neg_mean), 0.0)
        out_ref[...] = jnp.zeros_like(out_ref) + loss


def kernel(embeddings, labels):
    batch, dim = embeddings.shape
    block_rows = 128
    nsteps = batch // block_rows
    lab_row = labels.reshape(1, batch).astype(jnp.int32)
    lab_col = labels.reshape(batch, 1).astype(jnp.int32)

    out = pl.pallas_call(
        functools.partial(_hist_body, block_rows=block_rows, batch=batch),
        grid=(nsteps,),
        in_specs=[
            pl.BlockSpec((batch, dim), lambda i: (0, 0)),
            pl.BlockSpec((1, batch), lambda i: (0, 0)),
            pl.BlockSpec((batch, 1), lambda i: (0, 0)),
        ],
        out_specs=pl.BlockSpec((1, 128), lambda i: (0, 0)),
        out_shape=jax.ShapeDtypeStruct((1, 128), jnp.float32),
        scratch_shapes=[
            pltpu.VMEM((batch, dim), jnp.float32),
            pltpu.VMEM((8, 128), jnp.float32),
        ],
        compiler_params=pltpu.CompilerParams(
            dimension_semantics=("arbitrary",),
        ),
    )(embeddings, lab_row, lab_col)
    return out[0, 0]
